# baseline (speedup 1.0000x reference)
import functools

import jax
import jax.numpy as jnp
from jax import lax
from jax.experimental import pallas as pl
from jax.experimental.pallas import tpu as pltpu
from jax.experimental.pallas import tpu_sc as plsc

PAD = -100
_BATCH = 4
_SEQ = 8192
_NC, _NS, _L = 1, 16, 16
_NW = _NC * _NS
_BLK = _SEQ // _NW
_TILE = 128

_mesh = plsc.VectorSubcoreMesh(core_axis_name="c", subcore_axis_name="s",
                               num_cores=_NC)


@functools.partial(
    pl.kernel,
    out_type=(
        jax.ShapeDtypeStruct((_BATCH, _SEQ), jnp.int32),
        jax.ShapeDtypeStruct((_BATCH, _SEQ), jnp.int32),
        jax.ShapeDtypeStruct((_BATCH, _SEQ), jnp.int32),
    ),
    mesh=_mesh,
    scratch_types=[
        pltpu.VMEM((_BATCH, _BLK + _TILE), jnp.int32),
        pltpu.VMEM((_BATCH, _BLK), jnp.int32),
        pltpu.VMEM((_BATCH, _BLK), jnp.int32),
        pltpu.SemaphoreType.DMA,
        pltpu.SemaphoreType.DMA,
    ],
)
def _sc_process(in_hbm, masked_hbm, clm_hbm, den_hbm, buf, pad_buf, shift_buf,
                sem_in, sem_out):
    wid = lax.axis_index("s") * _NC + lax.axis_index("c")
    col = wid * _BLK
    is_last = wid == _NW - 1

    @pl.when(jnp.logical_not(is_last))
    def _():
        pltpu.async_copy(in_hbm.at[:, pl.ds(col, _BLK + _TILE)], buf,
                         sem_in).wait()

    @pl.when(is_last)
    def _():
        for r in range(_BATCH):
            buf[r, pl.ds(_BLK, _L)] = jnp.full((_L,), PAD, jnp.int32)
        pltpu.async_copy(in_hbm.at[:, pl.ds(col, _BLK)],
                         buf.at[:, pl.ds(0, _BLK)], sem_in).wait()

    pltpu.async_copy(buf.at[:, pl.ds(0, _BLK)],
                     masked_hbm.at[:, pl.ds(col, _BLK)], sem_out)

    is_suffix = wid >= _NW // 2

    @pl.when(is_suffix)
    def _():
        pltpu.async_copy(buf.at[:, pl.ds(0, _BLK)],
                         den_hbm.at[:, pl.ds(col, _BLK)], sem_out)

    @pl.when(jnp.logical_not(is_suffix))
    def _():
        def fill(j, carry):
            for r in range(_BATCH):
                pad_buf[r, pl.ds(j * _L, _L)] = jnp.full((_L,), PAD, jnp.int32)
            return carry

        lax.fori_loop(0, _BLK // _L, fill, 0, unroll=4)
        pltpu.async_copy(pad_buf, den_hbm.at[:, pl.ds(col, _BLK)], sem_out)

    lanes = lax.iota(jnp.int32, _L)
    roll_idx = lax.rem(lanes + 1, _L)
    not_last_lane = lanes < _L - 1

    def _roll1(v):
        return lax.gather(
            v, roll_idx[:, None],
            lax.GatherDimensionNumbers(offset_dims=(),
                                       collapsed_slice_dims=(0,),
                                       start_index_map=(0,)),
            (1,), mode=lax.GatherScatterMode.PROMISE_IN_BOUNDS)

    def shift(j, carry):
        for r in range(_BATCH):
            a = buf[r, pl.ds(j * _L, _L)]
            b = buf[r, pl.ds(j * _L + _L, _L)]
            shift_buf[r, pl.ds(j * _L, _L)] = jnp.where(
                not_last_lane, _roll1(a), _roll1(b))
        return carry

    lax.fori_loop(0, _BLK // _L, shift, 0, unroll=4)
    pltpu.async_copy(shift_buf, clm_hbm.at[:, pl.ds(col, _BLK)], sem_out)

    for _ in range(3):
        pltpu.make_async_copy(shift_buf, clm_hbm.at[:, pl.ds(col, _BLK)],
                              sem_out).wait()


def kernel(input_ids):
    return _sc_process(input_ids)


# speedup vs baseline: 1.0282x; 1.0282x over previous
import functools

import jax
import jax.numpy as jnp
from jax import lax
from jax.experimental import pallas as pl
from jax.experimental.pallas import tpu as pltpu
from jax.experimental.pallas import tpu_sc as plsc

PAD = -100
_BATCH = 4
_SEQ = 8192
_NC, _NS, _L = 1, 16, 16
_NW = _NC * _NS
_BLK = _SEQ // _NW
_TILE = 128

_mesh = plsc.VectorSubcoreMesh(core_axis_name="c", subcore_axis_name="s",
                               num_cores=_NC)


@functools.partial(
    pl.kernel,
    out_type=(
        jax.ShapeDtypeStruct((_BATCH, _SEQ), jnp.int32),
        jax.ShapeDtypeStruct((_BATCH, _SEQ), jnp.int32),
        jax.ShapeDtypeStruct((_BATCH, _SEQ), jnp.int32),
    ),
    mesh=_mesh,
    scratch_types=[
        pltpu.VMEM((_BATCH, _BLK + _TILE), jnp.int32),
        pltpu.VMEM((_BATCH, _BLK), jnp.int32),
        pltpu.VMEM((_BATCH, _BLK), jnp.int32),
        pltpu.SemaphoreType.DMA,
        pltpu.SemaphoreType.DMA,
    ],
)
def _sc_process(in_hbm, masked_hbm, clm_hbm, den_hbm, buf, pad_buf, shift_buf,
                sem_in, sem_out):
    wid = lax.axis_index("s") * _NC + lax.axis_index("c")
    col = wid * _BLK
    is_last = wid == _NW - 1

    @pl.when(jnp.logical_not(is_last))
    def _():
        pltpu.async_copy(in_hbm.at[:, pl.ds(col, _BLK + _TILE)], buf,
                         sem_in).wait()

    @pl.when(is_last)
    def _():
        for r in range(_BATCH):
            buf[r, pl.ds(_BLK, _L)] = jnp.full((_L,), PAD, jnp.int32)
        pltpu.async_copy(in_hbm.at[:, pl.ds(col, _BLK)],
                         buf.at[:, pl.ds(0, _BLK)], sem_in).wait()

    pltpu.async_copy(buf.at[:, pl.ds(0, _BLK)],
                     masked_hbm.at[:, pl.ds(col, _BLK)], sem_out)

    is_suffix = wid >= _NW // 2

    @pl.when(is_suffix)
    def _():
        pltpu.async_copy(buf.at[:, pl.ds(0, _BLK)],
                         den_hbm.at[:, pl.ds(col, _BLK)], sem_out)

    @pl.when(jnp.logical_not(is_suffix))
    def _():
        def fill(j, carry):
            for r in range(_BATCH):
                pad_buf[r, pl.ds(j * _L, _L)] = jnp.full((_L,), PAD, jnp.int32)
            return carry

        lax.fori_loop(0, _BLK // _L, fill, 0)
        pltpu.async_copy(pad_buf, den_hbm.at[:, pl.ds(col, _BLK)], sem_out)

    lanes = lax.iota(jnp.int32, _L)
    roll_idx = lax.rem(lanes + 1, _L)
    not_last_lane = lanes < _L - 1

    def _roll1(v):
        return lax.gather(
            v, roll_idx[:, None],
            lax.GatherDimensionNumbers(offset_dims=(),
                                       collapsed_slice_dims=(0,),
                                       start_index_map=(0,)),
            (1,), mode=lax.GatherScatterMode.PROMISE_IN_BOUNDS)

    def shift(j, carry):
        for r in range(_BATCH):
            a = buf[r, pl.ds(j * _L, _L)]
            b = buf[r, pl.ds(j * _L + _L, _L)]
            shift_buf[r, pl.ds(j * _L, _L)] = jnp.where(
                not_last_lane, _roll1(a), _roll1(b))
        return carry

    lax.fori_loop(0, _BLK // _L, shift, 0)
    pltpu.async_copy(shift_buf, clm_hbm.at[:, pl.ds(col, _BLK)], sem_out)

    for _ in range(3):
        pltpu.make_async_copy(shift_buf, clm_hbm.at[:, pl.ds(col, _BLK)],
                              sem_out).wait()


def kernel(input_ids):
    return _sc_process(input_ids)
